# baseline (speedup 1.0000x reference)
import functools

import jax
import jax.numpy as jnp
from jax import lax
from jax.experimental import pallas as pl
from jax.experimental.pallas import tpu as pltpu
from jax.experimental.pallas import tpu_sc as plsc

NC = 2
NS = 16
NW = NC * NS


def _sc_gather(user_table, item_table, user, item):
    B = user.shape[0]
    D = user_table.shape[1]
    b_per_w = B // NW
    mesh = plsc.VectorSubcoreMesh(core_axis_name="c", subcore_axis_name="s")

    @functools.partial(
        pl.kernel,
        mesh=mesh,
        out_type=(
            jax.ShapeDtypeStruct((B, D), jnp.float32),
            jax.ShapeDtypeStruct((B, D), jnp.float32),
        ),
        scratch_types=[
            pltpu.VMEM((b_per_w,), jnp.int32),
            pltpu.VMEM((b_per_w, D), jnp.float32),
            pltpu.SemaphoreType.DMA,
        ],
    )
    def gather_kernel(ut_hbm, it_hbm, uidx_hbm, iidx_hbm, uo_hbm, io_hbm,
                      idx_v, rows_v, sem):
        wid = lax.axis_index("s") * NC + lax.axis_index("c")
        base = wid * b_per_w
        pltpu.sync_copy(uidx_hbm.at[pl.ds(base, b_per_w)], idx_v)
        pltpu.async_copy(ut_hbm.at[idx_v], rows_v, sem).wait()
        pltpu.sync_copy(rows_v, uo_hbm.at[pl.ds(base, b_per_w)])
        pltpu.sync_copy(iidx_hbm.at[pl.ds(base, b_per_w)], idx_v)
        pltpu.async_copy(it_hbm.at[idx_v], rows_v, sem).wait()
        pltpu.sync_copy(rows_v, io_hbm.at[pl.ds(base, b_per_w)])

    return gather_kernel(user_table, item_table, user, item)


def _mlp_body(ue_ref, ie_ref, w1a_ref, w1b_ref, b1_ref, w2_ref, b2_ref,
              wfa_ref, wfb_ref, bf_ref, out_ref):
    ue = ue_ref[...]
    ie = ie_ref[...]
    h1 = jnp.dot(ue, w1a_ref[...], preferred_element_type=jnp.float32)
    h1 += jnp.dot(ie, w1b_ref[...], preferred_element_type=jnp.float32)
    h1 = jnp.maximum(h1 + b1_ref[...], 0.0)
    h2 = jnp.dot(h1, w2_ref[...], preferred_element_type=jnp.float32)
    h2 = jnp.maximum(h2 + b2_ref[...], 0.0)
    gmf = ue * ie
    out = jnp.sum(gmf * wfa_ref[...], axis=1) + jnp.sum(h2 * wfb_ref[...], axis=1)
    out_ref[...] = out + bf_ref[0]


def _tc_mlp(ue, ie, W1, b1, W2, b2, Wf, bf, block_b=1024):
    B, D = ue.shape
    H = W1.shape[0]
    w1a = W1[:, :D].T
    w1b = W1[:, D:].T
    w2 = W2.T
    D2 = w2.shape[1]
    wfa = Wf[:, :D]
    wfb = Wf[:, D:]
    b1r = b1.reshape(1, H)
    b2r = b2.reshape(1, D2)

    grid = (B // block_b,)
    return pl.pallas_call(
        _mlp_body,
        grid=grid,
        in_specs=[
            pl.BlockSpec((block_b, D), lambda i: (i, 0)),
            pl.BlockSpec((block_b, D), lambda i: (i, 0)),
            pl.BlockSpec((D, H), lambda i: (0, 0)),
            pl.BlockSpec((D, H), lambda i: (0, 0)),
            pl.BlockSpec((1, H), lambda i: (0, 0)),
            pl.BlockSpec((H, D2), lambda i: (0, 0)),
            pl.BlockSpec((1, D2), lambda i: (0, 0)),
            pl.BlockSpec((1, D), lambda i: (0, 0)),
            pl.BlockSpec((1, D2), lambda i: (0, 0)),
            pl.BlockSpec((1,), lambda i: (0,)),
        ],
        out_specs=pl.BlockSpec((block_b,), lambda i: (i,)),
        out_shape=jax.ShapeDtypeStruct((B,), jnp.float32),
    )(ue, ie, w1a, w1b, b1r, w2, b2r, wfa, wfb, bf)


@jax.jit
def kernel(user, item, user_table, item_table, W1, b1, W2, b2, Wf, bf):
    ue, ie = _sc_gather(user_table, item_table, user, item)
    return _tc_mlp(ue, ie, W1, b1, W2, b2, Wf, bf)


# speedup vs baseline: 1.0054x; 1.0054x over previous
import functools

import jax
import jax.numpy as jnp
from jax import lax
from jax.experimental import pallas as pl
from jax.experimental.pallas import tpu as pltpu
from jax.experimental.pallas import tpu_sc as plsc

NC = 2
NS = 16
NW = NC * NS


def _sc_gather(user_table, item_table, user, item):
    B = user.shape[0]
    D = user_table.shape[1]
    b_per_w = B // NW
    mesh = plsc.VectorSubcoreMesh(core_axis_name="c", subcore_axis_name="s")

    @functools.partial(
        pl.kernel,
        mesh=mesh,
        out_type=(
            jax.ShapeDtypeStruct((B, D), jnp.float32),
            jax.ShapeDtypeStruct((B, D), jnp.float32),
        ),
        scratch_types=[
            pltpu.VMEM((b_per_w,), jnp.int32),
            pltpu.VMEM((b_per_w, D), jnp.float32),
            pltpu.SemaphoreType.DMA,
        ],
    )
    def gather_kernel(ut_hbm, it_hbm, uidx_hbm, iidx_hbm, uo_hbm, io_hbm,
                      idx_v, rows_v, sem):
        wid = lax.axis_index("s") * NC + lax.axis_index("c")
        base = wid * b_per_w
        pltpu.sync_copy(uidx_hbm.at[pl.ds(base, b_per_w)], idx_v)
        pltpu.async_copy(ut_hbm.at[idx_v], rows_v, sem).wait()
        pltpu.sync_copy(rows_v, uo_hbm.at[pl.ds(base, b_per_w)])
        pltpu.sync_copy(iidx_hbm.at[pl.ds(base, b_per_w)], idx_v)
        pltpu.async_copy(it_hbm.at[idx_v], rows_v, sem).wait()
        pltpu.sync_copy(rows_v, io_hbm.at[pl.ds(base, b_per_w)])

    return gather_kernel(user_table, item_table, user, item)


def _mlp_body(ue_ref, ie_ref, w1a_ref, w1b_ref, b1_ref, w2_ref, b2_ref,
              wfa_ref, wfb_ref, bf_ref, out_ref):
    ue = ue_ref[...]
    ie = ie_ref[...]
    h1 = jnp.dot(ue.astype(jnp.bfloat16), w1a_ref[...],
                 preferred_element_type=jnp.float32)
    h1 += jnp.dot(ie.astype(jnp.bfloat16), w1b_ref[...],
                  preferred_element_type=jnp.float32)
    h1 = jnp.maximum(h1 + b1_ref[...], 0.0)
    h2 = jnp.dot(h1.astype(jnp.bfloat16), w2_ref[...],
                 preferred_element_type=jnp.float32)
    h2 = jnp.maximum(h2 + b2_ref[...], 0.0)
    gmf = ue * ie
    out = jnp.sum(gmf * wfa_ref[...], axis=1) + jnp.sum(h2 * wfb_ref[...], axis=1)
    out_ref[...] = out + bf_ref[0]


def _tc_mlp(ue, ie, W1, b1, W2, b2, Wf, bf, block_b=1024):
    B, D = ue.shape
    H = W1.shape[0]
    w1a = W1[:, :D].T.astype(jnp.bfloat16)
    w1b = W1[:, D:].T.astype(jnp.bfloat16)
    w2 = W2.T.astype(jnp.bfloat16)
    D2 = w2.shape[1]
    wfa = Wf[:, :D]
    wfb = Wf[:, D:]
    b1r = b1.reshape(1, H)
    b2r = b2.reshape(1, D2)

    grid = (B // block_b,)
    return pl.pallas_call(
        _mlp_body,
        grid=grid,
        in_specs=[
            pl.BlockSpec((block_b, D), lambda i: (i, 0)),
            pl.BlockSpec((block_b, D), lambda i: (i, 0)),
            pl.BlockSpec((D, H), lambda i: (0, 0)),
            pl.BlockSpec((D, H), lambda i: (0, 0)),
            pl.BlockSpec((1, H), lambda i: (0, 0)),
            pl.BlockSpec((H, D2), lambda i: (0, 0)),
            pl.BlockSpec((1, D2), lambda i: (0, 0)),
            pl.BlockSpec((1, D), lambda i: (0, 0)),
            pl.BlockSpec((1, D2), lambda i: (0, 0)),
            pl.BlockSpec((1,), lambda i: (0,)),
        ],
        out_specs=pl.BlockSpec((block_b,), lambda i: (i,)),
        out_shape=jax.ShapeDtypeStruct((B,), jnp.float32),
    )(ue, ie, w1a, w1b, b1r, w2, b2r, wfa, wfb, bf)


@jax.jit
def kernel(user, item, user_table, item_table, W1, b1, W2, b2, Wf, bf):
    ue, ie = _sc_gather(user_table, item_table, user, item)
    return _tc_mlp(ue, ie, W1, b1, W2, b2, Wf, bf)


# speedup vs baseline: 1.0798x; 1.0740x over previous
import functools

import jax
import jax.numpy as jnp
from jax import lax
from jax.experimental import pallas as pl
from jax.experimental.pallas import tpu as pltpu
from jax.experimental.pallas import tpu_sc as plsc

NC = 2
NS = 16
NW = NC * NS


def _sc_gather(user_table, item_table, user, item):
    B = user.shape[0]
    D = user_table.shape[1]
    b_per_w = B // NW
    mesh = plsc.VectorSubcoreMesh(core_axis_name="c", subcore_axis_name="s")

    @functools.partial(
        pl.kernel,
        mesh=mesh,
        out_type=(
            jax.ShapeDtypeStruct((B, D), jnp.float32),
            jax.ShapeDtypeStruct((B, D), jnp.float32),
        ),
        scratch_types=[
            pltpu.VMEM((b_per_w,), jnp.int32),
            pltpu.VMEM((b_per_w, D), jnp.float32),
            pltpu.SemaphoreType.DMA,
        ],
    )
    def gather_kernel(ut_hbm, it_hbm, uidx_hbm, iidx_hbm, uo_hbm, io_hbm,
                      idx_v, rows_v, sem):
        wid = lax.axis_index("s") * NC + lax.axis_index("c")
        base = wid * b_per_w
        pltpu.sync_copy(uidx_hbm.at[pl.ds(base, b_per_w)], idx_v)
        pltpu.async_copy(ut_hbm.at[idx_v], rows_v, sem).wait()
        pltpu.sync_copy(rows_v, uo_hbm.at[pl.ds(base, b_per_w)])
        pltpu.sync_copy(iidx_hbm.at[pl.ds(base, b_per_w)], idx_v)
        pltpu.async_copy(it_hbm.at[idx_v], rows_v, sem).wait()
        pltpu.sync_copy(rows_v, io_hbm.at[pl.ds(base, b_per_w)])

    return gather_kernel(user_table, item_table, user, item)


def _mlp_body(ue_ref, ie_ref, w1a_ref, w1b_ref, b1_ref, w2_ref, b2_ref,
              wfa_ref, wfb_ref, bf_ref, out_ref):
    ue = ue_ref[...]
    ie = ie_ref[...]
    h1 = jnp.maximum(
        jnp.dot(ue.astype(jnp.bfloat16), w1a_ref[...],
                preferred_element_type=jnp.float32)
        + jnp.dot(ie.astype(jnp.bfloat16), w1b_ref[...],
                  preferred_element_type=jnp.float32)
        + b1_ref[...], 0.0)
    h2 = jnp.maximum(
        jnp.dot(h1.astype(jnp.bfloat16), w2_ref[...],
                preferred_element_type=jnp.float32)
        + b2_ref[...], 0.0)
    gmf = (ue * ie).astype(jnp.bfloat16)
    out = (jnp.dot(gmf, wfa_ref[...], preferred_element_type=jnp.float32)
           + jnp.dot(h2.astype(jnp.bfloat16), wfb_ref[...],
                     preferred_element_type=jnp.float32))
    out_ref[...] = out + bf_ref[0]


def _tc_mlp(ue, ie, W1, b1, W2, b2, Wf, bf, block_b=1024):
    B, D = ue.shape
    H = W1.shape[0]
    w1a = W1[:, :D].T.astype(jnp.bfloat16)
    w1b = W1[:, D:].T.astype(jnp.bfloat16)
    w2 = W2.T.astype(jnp.bfloat16)
    D2 = w2.shape[1]
    wfa = Wf[:, :D].T.astype(jnp.bfloat16)
    wfb = Wf[:, D:].T.astype(jnp.bfloat16)
    b1r = b1.reshape(1, H)
    b2r = b2.reshape(1, D2)

    grid = (B // block_b,)
    out = pl.pallas_call(
        _mlp_body,
        grid=grid,
        in_specs=[
            pl.BlockSpec((block_b, D), lambda i: (i, 0)),
            pl.BlockSpec((block_b, D), lambda i: (i, 0)),
            pl.BlockSpec((D, H), lambda i: (0, 0)),
            pl.BlockSpec((D, H), lambda i: (0, 0)),
            pl.BlockSpec((1, H), lambda i: (0, 0)),
            pl.BlockSpec((H, D2), lambda i: (0, 0)),
            pl.BlockSpec((1, D2), lambda i: (0, 0)),
            pl.BlockSpec((D, 1), lambda i: (0, 0)),
            pl.BlockSpec((D2, 1), lambda i: (0, 0)),
            pl.BlockSpec((1,), lambda i: (0,)),
        ],
        out_specs=pl.BlockSpec((block_b, 1), lambda i: (i, 0)),
        out_shape=jax.ShapeDtypeStruct((B, 1), jnp.float32),
    )(ue, ie, w1a, w1b, b1r, w2, b2r, wfa, wfb, bf)
    return out[:, 0]


@jax.jit
def kernel(user, item, user_table, item_table, W1, b1, W2, b2, Wf, bf):
    ue, ie = _sc_gather(user_table, item_table, user, item)
    return _tc_mlp(ue, ie, W1, b1, W2, b2, Wf, bf)


# speedup vs baseline: 1.1552x; 1.0699x over previous
import functools

import jax
import jax.numpy as jnp
from jax import lax
from jax.experimental import pallas as pl
from jax.experimental.pallas import tpu as pltpu
from jax.experimental.pallas import tpu_sc as plsc

NC = 2
NS = 16
NW = NC * NS


def _sc_gather(user_table, item_table, user, item):
    B = user.shape[0]
    D = user_table.shape[1]
    b_per_w = B // NW
    mesh = plsc.VectorSubcoreMesh(core_axis_name="c", subcore_axis_name="s")

    @functools.partial(
        pl.kernel,
        mesh=mesh,
        out_type=(
            jax.ShapeDtypeStruct((B, D), jnp.float32),
            jax.ShapeDtypeStruct((B, D), jnp.float32),
        ),
        scratch_types=[
            pltpu.VMEM((b_per_w,), jnp.int32),
            pltpu.VMEM((b_per_w, D), jnp.float32),
            pltpu.SemaphoreType.DMA,
        ],
    )
    def gather_kernel(ut_hbm, it_hbm, uidx_hbm, iidx_hbm, uo_hbm, io_hbm,
                      idx_v, rows_v, sem):
        wid = lax.axis_index("s") * NC + lax.axis_index("c")
        base = wid * b_per_w
        pltpu.sync_copy(uidx_hbm.at[pl.ds(base, b_per_w)], idx_v)
        pltpu.async_copy(ut_hbm.at[idx_v], rows_v, sem).wait()
        pltpu.sync_copy(rows_v, uo_hbm.at[pl.ds(base, b_per_w)])
        pltpu.sync_copy(iidx_hbm.at[pl.ds(base, b_per_w)], idx_v)
        pltpu.async_copy(it_hbm.at[idx_v], rows_v, sem).wait()
        pltpu.sync_copy(rows_v, io_hbm.at[pl.ds(base, b_per_w)])

    return gather_kernel(user_table, item_table, user, item)


def _mlp_body(ue_ref, ie_ref, w1a_ref, w1b_ref, b1_ref, w2_ref, b2_ref,
              wfa_ref, wfb_ref, bf_ref, out_ref):
    ue = ue_ref[...]
    ie = ie_ref[...]
    h1 = jnp.maximum(
        jnp.dot(ue.astype(jnp.bfloat16), w1a_ref[...],
                preferred_element_type=jnp.float32)
        + jnp.dot(ie.astype(jnp.bfloat16), w1b_ref[...],
                  preferred_element_type=jnp.float32)
        + b1_ref[...], 0.0)
    h2 = jnp.maximum(
        jnp.dot(h1.astype(jnp.bfloat16), w2_ref[...],
                preferred_element_type=jnp.float32)
        + b2_ref[...], 0.0)
    gmf = (ue * ie).astype(jnp.bfloat16)
    out = (jnp.dot(gmf, wfa_ref[...], preferred_element_type=jnp.float32)
           + jnp.dot(h2.astype(jnp.bfloat16), wfb_ref[...],
                     preferred_element_type=jnp.float32))
    out_ref[...] = out + bf_ref[0]


def _prep_weights(D, W1, b1, W2, b2, Wf):
    H = W1.shape[0]
    w1a = W1[:, :D].T.astype(jnp.bfloat16)
    w1b = W1[:, D:].T.astype(jnp.bfloat16)
    w2 = W2.T.astype(jnp.bfloat16)
    wfa = Wf[:, :D].T.astype(jnp.bfloat16)
    wfb = Wf[:, D:].T.astype(jnp.bfloat16)
    b1r = b1.reshape(1, H)
    b2r = b2.reshape(1, w2.shape[1])
    return w1a, w1b, b1r, w2, b2r, wfa, wfb


def _tc_mlp(ue, ie, w1a, w1b, b1r, w2, b2r, wfa, wfb, bf, block_b=2048):
    B, D = ue.shape
    H = w1a.shape[1]
    D2 = w2.shape[1]

    grid = (B // block_b,)
    out = pl.pallas_call(
        _mlp_body,
        grid=grid,
        in_specs=[
            pl.BlockSpec((block_b, D), lambda i: (i, 0)),
            pl.BlockSpec((block_b, D), lambda i: (i, 0)),
            pl.BlockSpec((D, H), lambda i: (0, 0)),
            pl.BlockSpec((D, H), lambda i: (0, 0)),
            pl.BlockSpec((1, H), lambda i: (0, 0)),
            pl.BlockSpec((H, D2), lambda i: (0, 0)),
            pl.BlockSpec((1, D2), lambda i: (0, 0)),
            pl.BlockSpec((D, 1), lambda i: (0, 0)),
            pl.BlockSpec((D2, 1), lambda i: (0, 0)),
            pl.BlockSpec((1,), lambda i: (0,)),
        ],
        out_specs=pl.BlockSpec((block_b, 1), lambda i: (i, 0)),
        out_shape=jax.ShapeDtypeStruct((B, 1), jnp.float32),
    )(ue, ie, w1a, w1b, b1r, w2, b2r, wfa, wfb, bf)
    return out


NCHUNKS = 2


@jax.jit
def kernel(user, item, user_table, item_table, W1, b1, W2, b2, Wf, bf):
    B = user.shape[0]
    D = user_table.shape[1]
    wp = _prep_weights(D, W1, b1, W2, b2, Wf)
    Bc = B // NCHUNKS
    embs = [
        _sc_gather(user_table, item_table,
                   lax.dynamic_slice_in_dim(user, c * Bc, Bc),
                   lax.dynamic_slice_in_dim(item, c * Bc, Bc))
        for c in range(NCHUNKS)
    ]
    outs = [_tc_mlp(ue, ie, *wp, bf) for ue, ie in embs]
    return jnp.concatenate(outs, axis=0)[:, 0]


# speedup vs baseline: 1.2713x; 1.1005x over previous
import functools

import jax
import jax.numpy as jnp
from jax import lax
from jax.experimental import pallas as pl
from jax.experimental.pallas import tpu as pltpu
from jax.experimental.pallas import tpu_sc as plsc

NC = 2
NS = 16
NW = NC * NS


def _sc_gather(user_table, item_table, user, item):
    B = user.shape[0]
    D = user_table.shape[1]
    b_per_w = B // NW
    mesh = plsc.VectorSubcoreMesh(core_axis_name="c", subcore_axis_name="s")

    @functools.partial(
        pl.kernel,
        mesh=mesh,
        out_type=(
            jax.ShapeDtypeStruct((B, D), jnp.float32),
            jax.ShapeDtypeStruct((B, D), jnp.float32),
        ),
        scratch_types=[
            pltpu.VMEM((b_per_w,), jnp.int32),
            pltpu.VMEM((b_per_w, D), jnp.float32),
            pltpu.SemaphoreType.DMA,
        ],
    )
    def gather_kernel(ut_hbm, it_hbm, uidx_hbm, iidx_hbm, uo_hbm, io_hbm,
                      idx_v, rows_v, sem):
        wid = lax.axis_index("s") * NC + lax.axis_index("c")
        base = wid * b_per_w
        pltpu.sync_copy(uidx_hbm.at[pl.ds(base, b_per_w)], idx_v)
        pltpu.async_copy(ut_hbm.at[idx_v], rows_v, sem).wait()
        pltpu.sync_copy(rows_v, uo_hbm.at[pl.ds(base, b_per_w)])
        pltpu.sync_copy(iidx_hbm.at[pl.ds(base, b_per_w)], idx_v)
        pltpu.async_copy(it_hbm.at[idx_v], rows_v, sem).wait()
        pltpu.sync_copy(rows_v, io_hbm.at[pl.ds(base, b_per_w)])

    return gather_kernel(user_table, item_table, user, item)


def _mlp_body(ue_ref, ie_ref, w1a_ref, w1b_ref, b1_ref, w2_ref, b2_ref,
              wfa_ref, wfb_ref, bf_ref, out_ref):
    ue = ue_ref[...]
    ie = ie_ref[...]
    h1 = jnp.maximum(
        jnp.dot(ue.astype(jnp.bfloat16), w1a_ref[...],
                preferred_element_type=jnp.float32)
        + jnp.dot(ie.astype(jnp.bfloat16), w1b_ref[...],
                  preferred_element_type=jnp.float32)
        + b1_ref[...], 0.0)
    h2 = jnp.maximum(
        jnp.dot(h1.astype(jnp.bfloat16), w2_ref[...],
                preferred_element_type=jnp.float32)
        + b2_ref[...], 0.0)
    gmf = (ue * ie).astype(jnp.bfloat16)
    out = (jnp.dot(gmf, wfa_ref[...], preferred_element_type=jnp.float32)
           + jnp.dot(h2.astype(jnp.bfloat16), wfb_ref[...],
                     preferred_element_type=jnp.float32))
    out_ref[...] = (out + bf_ref[0]).T[None]


def _prep_weights(D, W1, b1, W2, b2, Wf):
    H = W1.shape[0]
    w1a = W1[:, :D].T.astype(jnp.bfloat16)
    w1b = W1[:, D:].T.astype(jnp.bfloat16)
    w2 = W2.T.astype(jnp.bfloat16)
    wfa = Wf[:, :D].T.astype(jnp.bfloat16)
    wfb = Wf[:, D:].T.astype(jnp.bfloat16)
    b1r = b1.reshape(1, H)
    b2r = b2.reshape(1, w2.shape[1])
    return w1a, w1b, b1r, w2, b2r, wfa, wfb


def _tc_mlp(ue, ie, w1a, w1b, b1r, w2, b2r, wfa, wfb, bf, block_b=2048):
    B, D = ue.shape
    H = w1a.shape[1]
    D2 = w2.shape[1]

    grid = (B // block_b,)
    out = pl.pallas_call(
        _mlp_body,
        grid=grid,
        in_specs=[
            pl.BlockSpec((block_b, D), lambda i: (i, 0)),
            pl.BlockSpec((block_b, D), lambda i: (i, 0)),
            pl.BlockSpec((D, H), lambda i: (0, 0)),
            pl.BlockSpec((D, H), lambda i: (0, 0)),
            pl.BlockSpec((1, H), lambda i: (0, 0)),
            pl.BlockSpec((H, D2), lambda i: (0, 0)),
            pl.BlockSpec((1, D2), lambda i: (0, 0)),
            pl.BlockSpec((D, 1), lambda i: (0, 0)),
            pl.BlockSpec((D2, 1), lambda i: (0, 0)),
            pl.BlockSpec((1,), lambda i: (0,)),
        ],
        out_specs=pl.BlockSpec((1, 1, block_b), lambda i: (i, 0, 0)),
        out_shape=jax.ShapeDtypeStruct((B // block_b, 1, block_b), jnp.float32),
    )(ue, ie, w1a, w1b, b1r, w2, b2r, wfa, wfb, bf)
    return out.reshape(B)


NCHUNKS = 2


@jax.jit
def kernel(user, item, user_table, item_table, W1, b1, W2, b2, Wf, bf):
    B = user.shape[0]
    D = user_table.shape[1]
    wp = _prep_weights(D, W1, b1, W2, b2, Wf)
    Bc = B // NCHUNKS
    embs = [
        _sc_gather(user_table, item_table,
                   lax.dynamic_slice_in_dim(user, c * Bc, Bc),
                   lax.dynamic_slice_in_dim(item, c * Bc, Bc))
        for c in range(NCHUNKS)
    ]
    outs = [_tc_mlp(ue, ie, *wp, bf) for ue, ie in embs]
    return jnp.concatenate(outs, axis=0)
